# baseline (speedup 1.0000x reference)
import functools
import jax
import jax.numpy as jnp
from jax import lax
from jax.experimental import pallas as pl
from jax.experimental.pallas import tpu as pltpu
from jax.experimental.pallas import tpu_sc as plsc

TOPK = 8
L = 16
RCHUNK = 8
NACC = 8
UNROLL = 32


def _bfly(v, iota, op):
    for s in (8, 4, 2, 1):
        perm = jnp.bitwise_xor(iota, s)
        v = op(v, v.at[perm].get(mode="promise_in_bounds"))
    return v


def _make_sc_rowmean(NROWS, S, NW):
    RPW = NROWS // NW
    NCH = RPW // RCHUNK
    NV = S // L
    inv_s = 1.0 / S

    mesh = plsc.VectorSubcoreMesh(core_axis_name="c", subcore_axis_name="s")

    @functools.partial(
        pl.kernel,
        mesh=mesh,
        out_type=jax.ShapeDtypeStruct((NROWS,), jnp.float32),
        scratch_types=[
            pltpu.VMEM((2, RCHUNK, S), jnp.float32),
            pltpu.VMEM((RPW,), jnp.float32),
            pltpu.SemaphoreType.DMA((2,)),
        ],
    )
    def sc_rowmean(x_hbm, out_hbm, buf, sums_v, sems):
        w = lax.axis_index("s") * 2 + lax.axis_index("c")
        base_row = w * RPW
        iota = lax.broadcasted_iota(jnp.int32, (L,), 0)

        def cp(g, slot):
            return pltpu.make_async_copy(
                x_hbm.at[pl.ds(base_row + g * RCHUNK, RCHUNK), :],
                buf.at[slot], sems.at[slot])

        cp(0, 0).start()
        cp(1, 1).start()

        def chunk_body(g, rv):
            slot = lax.rem(g, 2)
            cp(g, slot).wait()
            for r in range(RCHUNK):
                def jbody(j, accs):
                    base = j * (UNROLL * L)
                    out = list(accs)
                    for u in range(UNROLL):
                        v = buf[slot, r, pl.ds(base + u * L, L)]
                        out[u % NACC] = out[u % NACC] + v
                    return tuple(out)
                z = jnp.zeros((L,), jnp.float32)
                accs = lax.fori_loop(0, NV // UNROLL, jbody,
                                     tuple(z for _ in range(NACC)))
                a = ((accs[0] + accs[1]) + (accs[2] + accs[3])) + (
                    (accs[4] + accs[5]) + (accs[6] + accs[7]))
                acc = _bfly(a, iota, jnp.add)
                rv = jnp.where(iota == slot * RCHUNK + r, acc * inv_s, rv)

            @pl.when(lax.rem(g, 2) == 1)
            def _():
                sums_v[pl.ds((g // 2) * L, L)] = rv

            @pl.when(g + 2 < NCH)
            def _():
                cp(g + 2, lax.rem(g + 2, 2)).start()
            return rv

        lax.fori_loop(0, NCH, chunk_body, jnp.zeros((L,), jnp.float32))
        pltpu.sync_copy(sums_v, out_hbm.at[pl.ds(base_row, RPW)])

    return sc_rowmean


LANES = 128


def _tc_rowmean_body(x_ref, out_ref, part_scr):
    b = pl.program_id(0)
    nb = pl.num_programs(0)
    spatial = x_ref.shape[2]
    nsub = spatial // LANES
    xs = x_ref[0]
    acc = xs[:, 0:LANES]
    for j in range(1, nsub):
        acc = acc + xs[:, j * LANES:(j + 1) * LANES]
    part_scr[b] = acc

    @pl.when(b == nb - 1)
    def _():
        out_ref[...] = jnp.sum(part_scr[...], axis=2) * (1.0 / spatial)


def _tc_tail_body(xm_ref, gw_ref, gb_ref, eb_ref, probs_out, idx_out):
    xm = xm_ref[...]
    nrows, nexp = xm.shape[0], gw_ref.shape[0]
    logits = jax.lax.dot_general(
        xm, gw_ref[...], (((1,), (1,)), ((), ())),
        preferred_element_type=jnp.float32)
    logits = logits + gb_ref[...]
    logits = jnp.clip(logits, -10.0, 10.0)
    lb = logits + eb_ref[...]
    m = jnp.max(lb, axis=1, keepdims=True)
    e = jnp.exp(lb - m)
    p = e / jnp.sum(e, axis=1, keepdims=True)
    p = jnp.clip(p, 1e-06, 1.0)
    iota = jax.lax.broadcasted_iota(jnp.int32, (nrows, nexp), 1)
    vals, idxs = [], []
    for _ in range(TOPK):
        mk = jnp.max(p, axis=1, keepdims=True)
        ik = jnp.min(jnp.where(p == mk, iota, nexp), axis=1, keepdims=True)
        vals.append(mk)
        idxs.append(ik)
        p = jnp.where(iota == ik, -jnp.inf, p)
    tv = jnp.concatenate(vals, axis=1)
    ti = jnp.concatenate(idxs, axis=1)
    tv = tv / (jnp.sum(tv, axis=1, keepdims=True) + 1e-08)
    probs_out[...] = tv
    idx_out[...] = ti


def kernel(x, gate_w, gate_b, expert_bias):
    B, C, H, W = x.shape
    E = gate_w.shape[0]
    S = H * W
    B_SC = B // 2
    B_TC = B - B_SC
    xflat = x.reshape(B * C, S)
    xr = x.reshape(B, C, S)

    xm_sc = _make_sc_rowmean(B_SC * C, S, 32)(xflat)
    xm_tc = pl.pallas_call(
        _tc_rowmean_body,
        grid=(B_TC,),
        in_specs=[pl.BlockSpec((1, C, S), lambda b: (b + B_SC, 0, 0))],
        out_specs=pl.BlockSpec((B_TC, C), lambda b: (0, 0)),
        out_shape=jax.ShapeDtypeStruct((B_TC, C), jnp.float32),
        scratch_shapes=[pltpu.VMEM((B_TC, C, LANES), jnp.float32)],
    )(xr)
    xm = jnp.concatenate([xm_sc.reshape(B_SC, C), xm_tc], axis=0)

    gb = gate_b.reshape(1, E)
    eb = expert_bias.reshape(1, E)
    probs, idx = pl.pallas_call(
        _tc_tail_body,
        in_specs=[
            pl.BlockSpec((B, C), lambda: (0, 0)),
            pl.BlockSpec((E, C), lambda: (0, 0)),
            pl.BlockSpec((1, E), lambda: (0, 0)),
            pl.BlockSpec((1, E), lambda: (0, 0)),
        ],
        out_specs=[
            pl.BlockSpec((B, TOPK), lambda: (0, 0)),
            pl.BlockSpec((B, TOPK), lambda: (0, 0)),
        ],
        out_shape=[
            jax.ShapeDtypeStruct((B, TOPK), jnp.float32),
            jax.ShapeDtypeStruct((B, TOPK), jnp.int32),
        ],
    )(xm, gate_w, gb, eb)

    loss = jnp.zeros((), dtype=jnp.float32)
    return (probs, idx, loss)


# speedup vs baseline: 1.1872x; 1.1872x over previous
import functools
import jax
import jax.numpy as jnp
from jax import lax
from jax.experimental import pallas as pl
from jax.experimental.pallas import tpu as pltpu
from jax.experimental.pallas import tpu_sc as plsc

TOPK = 8
L = 16
RCHUNK = 8
NACC = 4
UNROLL = 32


def _bfly(v, iota, op):
    for s in (8, 4, 2, 1):
        perm = jnp.bitwise_xor(iota, s)
        v = op(v, v.at[perm].get(mode="promise_in_bounds"))
    return v


def _make_sc_rowmean(NROWS, S, NW):
    RPW = NROWS // NW
    NCH = RPW // RCHUNK
    NV = S // L
    inv_s = 1.0 / S

    mesh = plsc.VectorSubcoreMesh(core_axis_name="c", subcore_axis_name="s")

    @functools.partial(
        pl.kernel,
        mesh=mesh,
        out_type=jax.ShapeDtypeStruct((NROWS,), jnp.float32),
        scratch_types=[
            pltpu.VMEM((2, RCHUNK, S), jnp.float32),
            pltpu.VMEM((RPW,), jnp.float32),
            pltpu.SemaphoreType.DMA((2,)),
        ],
    )
    def sc_rowmean(x_hbm, out_hbm, buf, sums_v, sems):
        w = lax.axis_index("s") * 2 + lax.axis_index("c")
        base_row = w * RPW
        iota = lax.broadcasted_iota(jnp.int32, (L,), 0)

        def cp(g, slot):
            return pltpu.make_async_copy(
                x_hbm.at[pl.ds(base_row + g * RCHUNK, RCHUNK), :],
                buf.at[slot], sems.at[slot])

        cp(0, 0).start()
        cp(1, 1).start()

        def chunk_body(g, rv):
            slot = lax.rem(g, 2)
            cp(g, slot).wait()
            for r in range(RCHUNK):
                z = jnp.zeros((L,), jnp.float32)

                @plsc.parallel_loop(0, NV * L, step=NACC * L,
                                    unroll=UNROLL // NACC,
                                    carry=tuple(z for _ in range(NACC)))
                def accs(j, carry):
                    out = list(carry)
                    for u in range(NACC):
                        v = buf[slot, r, pl.ds(j + u * L, L)]
                        out[u] = out[u] + v
                    return tuple(out)

                a = (accs[0] + accs[1]) + (accs[2] + accs[3])
                acc = _bfly(a, iota, jnp.add)
                rv = jnp.where(iota == slot * RCHUNK + r, acc * inv_s, rv)

            @pl.when(lax.rem(g, 2) == 1)
            def _():
                sums_v[pl.ds((g // 2) * L, L)] = rv

            @pl.when(g + 2 < NCH)
            def _():
                cp(g + 2, lax.rem(g + 2, 2)).start()
            return rv

        lax.fori_loop(0, NCH, chunk_body, jnp.zeros((L,), jnp.float32))
        pltpu.sync_copy(sums_v, out_hbm.at[pl.ds(base_row, RPW)])

    return sc_rowmean


LANES = 128


def _tc_rowmean_body(x_ref, out_ref, part_scr):
    b = pl.program_id(0)
    nb = pl.num_programs(0)
    spatial = x_ref.shape[2]
    nsub = spatial // LANES
    xs = x_ref[0]
    acc = xs[:, 0:LANES]
    for j in range(1, nsub):
        acc = acc + xs[:, j * LANES:(j + 1) * LANES]
    part_scr[b] = acc

    @pl.when(b == nb - 1)
    def _():
        out_ref[...] = jnp.sum(part_scr[...], axis=2) * (1.0 / spatial)


def _tc_tail_body(xm_ref, gw_ref, gb_ref, eb_ref, probs_out, idx_out):
    xm = xm_ref[...]
    nrows, nexp = xm.shape[0], gw_ref.shape[0]
    logits = jax.lax.dot_general(
        xm, gw_ref[...], (((1,), (1,)), ((), ())),
        preferred_element_type=jnp.float32)
    logits = logits + gb_ref[...]
    logits = jnp.clip(logits, -10.0, 10.0)
    lb = logits + eb_ref[...]
    m = jnp.max(lb, axis=1, keepdims=True)
    e = jnp.exp(lb - m)
    p = e / jnp.sum(e, axis=1, keepdims=True)
    p = jnp.clip(p, 1e-06, 1.0)
    iota = jax.lax.broadcasted_iota(jnp.int32, (nrows, nexp), 1)
    vals, idxs = [], []
    for _ in range(TOPK):
        mk = jnp.max(p, axis=1, keepdims=True)
        ik = jnp.min(jnp.where(p == mk, iota, nexp), axis=1, keepdims=True)
        vals.append(mk)
        idxs.append(ik)
        p = jnp.where(iota == ik, -jnp.inf, p)
    tv = jnp.concatenate(vals, axis=1)
    ti = jnp.concatenate(idxs, axis=1)
    tv = tv / (jnp.sum(tv, axis=1, keepdims=True) + 1e-08)
    probs_out[...] = tv
    idx_out[...] = ti


def kernel(x, gate_w, gate_b, expert_bias):
    B, C, H, W = x.shape
    E = gate_w.shape[0]
    S = H * W
    xflat = x.reshape(B * C, S)

    xm = _make_sc_rowmean(B * C, S, 32)(xflat).reshape(B, C)

    gb = gate_b.reshape(1, E)
    eb = expert_bias.reshape(1, E)
    probs, idx = pl.pallas_call(
        _tc_tail_body,
        in_specs=[
            pl.BlockSpec((B, C), lambda: (0, 0)),
            pl.BlockSpec((E, C), lambda: (0, 0)),
            pl.BlockSpec((1, E), lambda: (0, 0)),
            pl.BlockSpec((1, E), lambda: (0, 0)),
        ],
        out_specs=[
            pl.BlockSpec((B, TOPK), lambda: (0, 0)),
            pl.BlockSpec((B, TOPK), lambda: (0, 0)),
        ],
        out_shape=[
            jax.ShapeDtypeStruct((B, TOPK), jnp.float32),
            jax.ShapeDtypeStruct((B, TOPK), jnp.int32),
        ],
    )(xm, gate_w, gb, eb)

    loss = jnp.zeros((), dtype=jnp.float32)
    return (probs, idx, loss)


# speedup vs baseline: 2.8372x; 2.3899x over previous
import functools
import jax
import jax.numpy as jnp
from jax import lax
from jax.experimental import pallas as pl
from jax.experimental.pallas import tpu as pltpu
from jax.experimental.pallas import tpu_sc as plsc

TOPK = 8
L = 16
LANES = 128


def _bfly(v, iota, op):
    for s in (8, 4, 2, 1):
        perm = jnp.bitwise_xor(iota, s)
        v = op(v, v.at[perm].get(mode="promise_in_bounds"))
    return v


def _tc_logits_body(x_ref, gw_ref, gb_ref, eb_ref, lg_out, part_scr):
    b = pl.program_id(0)
    nb = pl.num_programs(0)
    spatial = x_ref.shape[2]
    nsub = spatial // LANES
    xs = x_ref[0]
    acc = xs[:, 0:LANES]
    for j in range(1, nsub):
        acc = acc + xs[:, j * LANES:(j + 1) * LANES]
    part_scr[b] = acc

    @pl.when(b == nb - 1)
    def _():
        xm = jnp.sum(part_scr[...], axis=2) * (1.0 / spatial)
        logits = jax.lax.dot_general(
            xm, gw_ref[...], (((1,), (1,)), ((), ())),
            preferred_element_type=jnp.float32)
        logits = logits + gb_ref[...]
        logits = jnp.clip(logits, -10.0, 10.0)
        lg_out[...] = logits + eb_ref[...]


def _make_sc_router(B, E):
    NQ = E // L

    mesh = plsc.VectorSubcoreMesh(core_axis_name="c", subcore_axis_name="s")

    @functools.partial(
        pl.kernel,
        mesh=mesh,
        out_type=[
            jax.ShapeDtypeStruct((B, L), jnp.float32),
            jax.ShapeDtypeStruct((B, L), jnp.int32),
        ],
        scratch_types=[
            pltpu.VMEM((E,), jnp.float32),
            pltpu.VMEM((L,), jnp.float32),
            pltpu.VMEM((L,), jnp.int32),
        ],
    )
    def sc_router(lg_hbm, probs_hbm, idx_hbm, lbuf, pbuf, ibuf):
        w = lax.axis_index("s") * 2 + lax.axis_index("c")
        iota = lax.broadcasted_iota(jnp.int32, (L,), 0)

        pltpu.sync_copy(lg_hbm.at[w], lbuf)
        lg = [lbuf[pl.ds(q * L, L)] for q in range(NQ)]

        mv = jnp.maximum(jnp.maximum(lg[0], lg[1]),
                         jnp.maximum(lg[2], lg[3]))
        mxv = _bfly(mv, iota, jnp.maximum)
        eq = [jnp.exp(lg[q] - mxv) for q in range(NQ)]
        esv = _bfly((eq[0] + eq[1]) + (eq[2] + eq[3]), iota, jnp.add)
        p = [jnp.clip(e / esv, 1e-06, 1.0) for e in eq]

        topv = jnp.zeros((L,), jnp.float32)
        topi = jnp.zeros((L,), jnp.int32)
        psum = jnp.zeros((L,), jnp.float32)
        sel = list(lg)
        for k in range(TOPK):
            m = jnp.maximum(jnp.maximum(sel[0], sel[1]),
                            jnp.maximum(sel[2], sel[3]))
            mkv = _bfly(m, iota, jnp.maximum)
            cand = [jnp.where(sel[q] == mkv, iota + q * L, E)
                    for q in range(NQ)]
            cm = jnp.minimum(jnp.minimum(cand[0], cand[1]),
                             jnp.minimum(cand[2], cand[3]))
            civ = _bfly(cm, iota, jnp.minimum)
            hit = [iota + q * L == civ for q in range(NQ)]
            pv = jnp.where(hit[0], p[0], 0.0)
            for q in range(1, NQ):
                pv = pv + jnp.where(hit[q], p[q], 0.0)
            pkv = _bfly(pv, iota, jnp.add)
            topv = jnp.where(iota == k, pkv, topv)
            topi = jnp.where(iota == k, civ, topi)
            psum = psum + pkv
            for q in range(NQ):
                sel[q] = jnp.where(hit[q], -3.0e38, sel[q])

        pbuf[...] = topv / (psum + 1e-08)
        ibuf[...] = topi
        pltpu.sync_copy(pbuf, probs_hbm.at[w])
        pltpu.sync_copy(ibuf, idx_hbm.at[w])

    return sc_router


def kernel(x, gate_w, gate_b, expert_bias):
    B, C, H, W = x.shape
    E = gate_w.shape[0]
    S = H * W
    xr = x.reshape(B, C, S)
    gb = gate_b.reshape(1, E)
    eb = expert_bias.reshape(1, E)

    logits = pl.pallas_call(
        _tc_logits_body,
        grid=(B,),
        in_specs=[
            pl.BlockSpec((1, C, S), lambda b: (b, 0, 0)),
            pl.BlockSpec((E, C), lambda b: (0, 0)),
            pl.BlockSpec((1, E), lambda b: (0, 0)),
            pl.BlockSpec((1, E), lambda b: (0, 0)),
        ],
        out_specs=pl.BlockSpec((B, E), lambda b: (0, 0)),
        out_shape=jax.ShapeDtypeStruct((B, E), jnp.float32),
        scratch_shapes=[pltpu.VMEM((B, C, LANES), jnp.float32)],
    )(xr, gate_w, gb, eb)

    probs16, idx16 = _make_sc_router(B, E)(logits)
    probs = probs16[:, :TOPK]
    idx = idx16[:, :TOPK]
    loss = jnp.zeros((), dtype=jnp.float32)
    return (probs, idx, loss)
